# baseline (speedup 1.0000x reference)
import functools

import jax
import jax.numpy as jnp
from jax import lax
from jax.experimental import pallas as pl
from jax.experimental.pallas import tpu as pltpu
from jax.experimental.pallas import tpu_sc as plsc

N = 10000
E = 320000
H = 128
ED = 4
L = 5
G = 64

EPT = E // 16
C = 80
NCHUNK = EPT // C
NPAD = 10240
HH = H // 2
RPT = NPAD // 16
RCH = 128
NRCH = RPT // RCH

BLK = 1000
NBLK = N // BLK

_LN2 = 0.6931471805599453
_LOGC = (-1.7207799231e-02, 1.8497244068e-01, -8.5552603849e-01,
         2.2311249058e+00, -3.6488033354e+00, 4.2045130615e+00,
         -2.0990697238e+00)


def _softplus16(x):
    t = jnp.exp(x)
    y = 1.0 + t
    i = plsc.bitcast(y, jnp.int32)
    e = lax.shift_right_logical(i, 23).astype(jnp.float32) - 127.0
    m = plsc.bitcast(
        jnp.bitwise_or(jnp.bitwise_and(i, 0x007FFFFF), 0x3F800000),
        jnp.float32)
    p = jnp.full((16,), _LOGC[0], jnp.float32)
    for c in _LOGC[1:]:
        p = p * m + c
    r = e * _LN2 + p
    return jnp.where(x > 15.0, x, r)


def _sigmoid16(x):
    return 1.0 / (1.0 + jnp.exp(-x))


EPW = E // 16
CC = 80
NCC = EPW // CC
CE = 128
NCE = E // CE
FPT = 4
TW = N * FPT

_mesh = plsc.VectorSubcoreMesh(core_axis_name="c", subcore_axis_name="s")


def _make_cnt_kernel():
    scratch = [
        pltpu.VMEM((CC,), jnp.int32),
        pltpu.VMEM((1, N), jnp.float32),
    ]

    def body(dst_hbm, part_out, dstv, cnt):
        cid = lax.axis_index("c")
        sid = lax.axis_index("s")
        zero16 = jnp.zeros((16,), jnp.float32)
        ones16 = jnp.ones((16,), jnp.float32)
        zi16 = jnp.zeros((16,), jnp.int32)

        @pl.when(cid == 0)
        def _():
            def zf(r, _):
                cnt[0, pl.ds(r * 16, 16)] = zero16
                return 0
            lax.fori_loop(0, N // 16, zf, 0)

            base0 = sid * EPW

            def chunk(t, _):
                pltpu.sync_copy(dst_hbm.at[pl.ds(base0 + t * CC, CC)], dstv)
                for j in range(CC // 16):
                    d16 = dstv[pl.ds(j * 16, 16)]
                    plsc.addupdate_scatter(cnt, [zi16, d16], ones16)
                return 0
            lax.fori_loop(0, NCC, chunk, 0)
            pltpu.sync_copy(cnt, part_out.at[sid])

    return pl.kernel(body,
                     out_type=jax.ShapeDtypeStruct((16, 1, N), jnp.float32),
                     mesh=_mesh, scratch_types=scratch,
                     compiler_params=pltpu.CompilerParams(
                         needs_layout_passes=False))


_cnt_kernel = _make_cnt_kernel()


def _make_edge_kernel():
    scratch = [
        pltpu.VMEM((CE,), jnp.int32),
        pltpu.VMEM((CE,), jnp.int32),
        pltpu.VMEM((4, CE), jnp.float32),
        pltpu.VMEM((1, TW), jnp.int32),
        pltpu.VMEM((1, TW), jnp.int32),
        pltpu.VMEM((1, TW), jnp.float32),
        pltpu.VMEM((8, 16), jnp.float32),
    ]

    def body(dst_hbm, src_hbm, eat_hbm, ut_hbm, vt_hbm, wet_hbm,
             agg_out, dstv, srcv, eat, utab, vtab, agg, wet):
        cid = lax.axis_index("c")
        sid = lax.axis_index("s")
        w = cid * 16 + sid

        zero16 = jnp.zeros((16,), jnp.float32)
        zi16 = jnp.zeros((16,), jnp.int32)

        pltpu.sync_copy(ut_hbm.at[w], utab)
        pltpu.sync_copy(vt_hbm.at[w], vtab)
        pltpu.sync_copy(wet_hbm.at[w], wet)

        def zf(r, _):
            agg[0, pl.ds(r * 16, 16)] = zero16
            return 0
        lax.fori_loop(0, TW // 16, zf, 0)

        wf = [[wet[k, pl.ds(0, 16)][c] for c in range(FPT)] for k in range(4)]
        ws = [[wet[4 + k, pl.ds(0, 16)][c] for c in range(FPT)]
              for k in range(4)]

        def chunk(t, _):
            b = t * CE
            pltpu.sync_copy(dst_hbm.at[pl.ds(b, CE)], dstv)
            pltpu.sync_copy(src_hbm.at[pl.ds(b, CE)], srcv)
            pltpu.sync_copy(eat_hbm.at[pl.ds(0, 4), pl.ds(b, CE)], eat)
            for j in range(CE // 16):
                o = j * 16
                d16 = dstv[pl.ds(o, 16)]
                s16 = srcv[pl.ds(o, 16)]
                ea = [eat[k, pl.ds(o, 16)] for k in range(4)]
                d4 = d16 * FPT
                s4 = s16 * FPT
                for c in range(FPT):
                    ug = plsc.load_gather(utab, [zi16, d4 + c])
                    vg = plsc.load_gather(vtab, [zi16, s4 + c])
                    ufi = lax.shift_right_arithmetic(ug, 16)
                    usi = lax.shift_right_arithmetic(lax.shift_left(ug, 16),
                                                     16)
                    vfi = lax.shift_right_arithmetic(vg, 16)
                    vsi = lax.shift_right_arithmetic(lax.shift_left(vg, 16),
                                                     16)
                    lf = (ufi + vfi).astype(jnp.float32) * 9.765625e-4
                    ls = (usi + vsi).astype(jnp.float32) * 9.765625e-4
                    for k in range(4):
                        lf = lf + ea[k] * wf[k][c]
                        ls = ls + ea[k] * ws[k][c]
                    msg = _sigmoid16(lf) * _softplus16(ls)
                    plsc.addupdate_scatter(agg, [zi16, d4 + c], msg)
            return 0
        lax.fori_loop(0, NCE, chunk, 0)

        pltpu.sync_copy(agg, agg_out.at[w])

    return pl.kernel(body,
                     out_type=jax.ShapeDtypeStruct((32, 1, TW), jnp.float32),
                     mesh=_mesh, scratch_types=scratch,
                     compiler_params=pltpu.CompilerParams(
                         needs_layout_passes=False))


_edge_kernel = _make_edge_kernel()


def _k0_body(x_ref, win_ref, bin_ref, wd_ref, bd_ref, wsrc_ref,
             h_ref, u_ref, v_ref):
    h = jnp.dot(x_ref[...], win_ref[...],
                preferred_element_type=jnp.float32) + bin_ref[...]
    h_ref[...] = h
    u_ref[...] = jnp.dot(h, wd_ref[...],
                         preferred_element_type=jnp.float32) + bd_ref[...]
    v_ref[...] = jnp.dot(h, wsrc_ref[...], preferred_element_type=jnp.float32)


def _k0(x, win, binb, wd, bd, wsrc):
    return pl.pallas_call(
        _k0_body,
        grid=(NBLK,),
        in_specs=[
            pl.BlockSpec((BLK, H), lambda i: (i, 0)),
            pl.BlockSpec((H, H), lambda i: (0, 0)),
            pl.BlockSpec((1, H), lambda i: (0, 0)),
            pl.BlockSpec((H, 2 * H), lambda i: (0, 0)),
            pl.BlockSpec((1, 2 * H), lambda i: (0, 0)),
            pl.BlockSpec((H, 2 * H), lambda i: (0, 0)),
        ],
        out_specs=[
            pl.BlockSpec((BLK, H), lambda i: (i, 0)),
            pl.BlockSpec((BLK, 2 * H), lambda i: (i, 0)),
            pl.BlockSpec((BLK, 2 * H), lambda i: (i, 0)),
        ],
        out_shape=[
            jax.ShapeDtypeStruct((N, H), jnp.float32),
            jax.ShapeDtypeStruct((N, 2 * H), jnp.float32),
            jax.ShapeDtypeStruct((N, 2 * H), jnp.float32),
        ],
    )(x, win, binb, wd, bd, wsrc)


def _kstats0_body(h_ref, agg_ref, cntp_ref, hmid_ref, s1_ref, s2_ref,
                  cntc_ref):
    i = pl.program_id(0)
    cs = cntp_ref[0]
    c = jnp.maximum(jnp.sum(cs, axis=0), 1.0).reshape(BLK, 1)
    cntc_ref[...] = c
    hm = h_ref[...] + agg_ref[...] / c
    hmid_ref[...] = hm

    @pl.when(i == 0)
    def _():
        s1_ref[...] = jnp.sum(hm, axis=0, keepdims=True)
        s2_ref[...] = jnp.sum(hm * hm, axis=0, keepdims=True)

    @pl.when(i != 0)
    def _():
        s1_ref[...] += jnp.sum(hm, axis=0, keepdims=True)
        s2_ref[...] += jnp.sum(hm * hm, axis=0, keepdims=True)


def _kstats0(h, agg, cntp):
    return pl.pallas_call(
        _kstats0_body,
        grid=(NBLK,),
        in_specs=[
            pl.BlockSpec((BLK, H), lambda i: (i, 0)),
            pl.BlockSpec((BLK, H), lambda i: (i, 0)),
            pl.BlockSpec((1, 16, BLK), lambda i: (i, 0, 0)),
        ],
        out_specs=[
            pl.BlockSpec((BLK, H), lambda i: (i, 0)),
            pl.BlockSpec((1, H), lambda i: (0, 0)),
            pl.BlockSpec((1, H), lambda i: (0, 0)),
            pl.BlockSpec((BLK, 1), lambda i: (i, 0)),
        ],
        out_shape=[
            jax.ShapeDtypeStruct((N, H), jnp.float32),
            jax.ShapeDtypeStruct((1, H), jnp.float32),
            jax.ShapeDtypeStruct((1, H), jnp.float32),
            jax.ShapeDtypeStruct((N, 1), jnp.float32),
        ],
    )(h, agg, cntp)


def _kstats_body(h_ref, agg_ref, cnt_ref, hmid_ref, s1_ref, s2_ref):
    hm = h_ref[...] + agg_ref[...] / cnt_ref[...]
    hmid_ref[...] = hm
    i = pl.program_id(0)

    @pl.when(i == 0)
    def _():
        s1_ref[...] = jnp.sum(hm, axis=0, keepdims=True)
        s2_ref[...] = jnp.sum(hm * hm, axis=0, keepdims=True)

    @pl.when(i != 0)
    def _():
        s1_ref[...] += jnp.sum(hm, axis=0, keepdims=True)
        s2_ref[...] += jnp.sum(hm * hm, axis=0, keepdims=True)


def _kstats(h, agg, cnt):
    return pl.pallas_call(
        _kstats_body,
        grid=(NBLK,),
        in_specs=[
            pl.BlockSpec((BLK, H), lambda i: (i, 0)),
            pl.BlockSpec((BLK, H), lambda i: (i, 0)),
            pl.BlockSpec((BLK, 1), lambda i: (i, 0)),
        ],
        out_specs=[
            pl.BlockSpec((BLK, H), lambda i: (i, 0)),
            pl.BlockSpec((1, H), lambda i: (0, 0)),
            pl.BlockSpec((1, H), lambda i: (0, 0)),
        ],
        out_shape=[
            jax.ShapeDtypeStruct((N, H), jnp.float32),
            jax.ShapeDtypeStruct((1, H), jnp.float32),
            jax.ShapeDtypeStruct((1, H), jnp.float32),
        ],
    )(h, agg, cnt)


def _bn_relu(hmid, s1, s2, g, b):
    mu = s1 / float(N)
    var = s2 / float(N) - mu * mu
    return jnp.maximum(g * (hmid - mu) * lax.rsqrt(var + 1e-5) + b, 0.0)


def _kapply_body(hmid_ref, s1_ref, s2_ref, g_ref, b_ref, wd_ref, bd_ref,
                 wsrc_ref, hn_ref, u_ref, v_ref):
    hn = _bn_relu(hmid_ref[...], s1_ref[...], s2_ref[...], g_ref[...],
                  b_ref[...])
    hn_ref[...] = hn
    u_ref[...] = jnp.dot(hn, wd_ref[...],
                         preferred_element_type=jnp.float32) + bd_ref[...]
    v_ref[...] = jnp.dot(hn, wsrc_ref[...], preferred_element_type=jnp.float32)


def _kapply(hmid, s1, s2, g, b, wd, bd, wsrc):
    return pl.pallas_call(
        _kapply_body,
        grid=(NBLK,),
        in_specs=[
            pl.BlockSpec((BLK, H), lambda i: (i, 0)),
            pl.BlockSpec((1, H), lambda i: (0, 0)),
            pl.BlockSpec((1, H), lambda i: (0, 0)),
            pl.BlockSpec((1, H), lambda i: (0, 0)),
            pl.BlockSpec((1, H), lambda i: (0, 0)),
            pl.BlockSpec((H, 2 * H), lambda i: (0, 0)),
            pl.BlockSpec((1, 2 * H), lambda i: (0, 0)),
            pl.BlockSpec((H, 2 * H), lambda i: (0, 0)),
        ],
        out_specs=[
            pl.BlockSpec((BLK, H), lambda i: (i, 0)),
            pl.BlockSpec((BLK, 2 * H), lambda i: (i, 0)),
            pl.BlockSpec((BLK, 2 * H), lambda i: (i, 0)),
        ],
        out_shape=[
            jax.ShapeDtypeStruct((N, H), jnp.float32),
            jax.ShapeDtypeStruct((N, 2 * H), jnp.float32),
            jax.ShapeDtypeStruct((N, 2 * H), jnp.float32),
        ],
    )(hmid, s1, s2, g, b, wd, bd, wsrc)


def _kpool_body(hmid_ref, s1_ref, s2_ref, g_ref, b_ref, batch_ref,
                w1_ref, b1_ref, w2_ref, b2_ref, w3_ref, b3_ref,
                pooled_ref, gcnt_ref, out_ref):
    hn = _bn_relu(hmid_ref[...], s1_ref[...], s2_ref[...], g_ref[...],
                  b_ref[...])
    bt = batch_ref[...]
    gids = lax.broadcasted_iota(jnp.int32, (G, BLK), 0)
    m = (gids == bt[:, 0][None, :]).astype(jnp.float32)
    pp = jnp.dot(m, hn, preferred_element_type=jnp.float32)
    gc = jnp.sum(m, axis=1, keepdims=True)
    i = pl.program_id(0)

    @pl.when(i == 0)
    def _():
        pooled_ref[...] = pp
        gcnt_ref[...] = gc

    @pl.when(i != 0)
    def _():
        pooled_ref[...] += pp
        gcnt_ref[...] += gc

    @pl.when(i == NBLK - 1)
    def _():
        p = pooled_ref[...] / jnp.maximum(gcnt_ref[...], 1.0)
        o = jnp.maximum(jnp.dot(p, w1_ref[...],
                                preferred_element_type=jnp.float32)
                        + b1_ref[...], 0.0)
        o = jnp.maximum(jnp.dot(o, w2_ref[...],
                                preferred_element_type=jnp.float32)
                        + b2_ref[...], 0.0)
        out_ref[...] = jnp.dot(o, w3_ref[...],
                               preferred_element_type=jnp.float32) + b3_ref[...]


def _kpool(hmid, s1, s2, g, b, batch2, w1, b1, w2, b2, w3, b3):
    outs = pl.pallas_call(
        _kpool_body,
        grid=(NBLK,),
        in_specs=[
            pl.BlockSpec((BLK, H), lambda i: (i, 0)),
            pl.BlockSpec((1, H), lambda i: (0, 0)),
            pl.BlockSpec((1, H), lambda i: (0, 0)),
            pl.BlockSpec((1, H), lambda i: (0, 0)),
            pl.BlockSpec((1, H), lambda i: (0, 0)),
            pl.BlockSpec((BLK, 1), lambda i: (i, 0)),
            pl.BlockSpec((H, H), lambda i: (0, 0)),
            pl.BlockSpec((1, H), lambda i: (0, 0)),
            pl.BlockSpec((H, H), lambda i: (0, 0)),
            pl.BlockSpec((1, H), lambda i: (0, 0)),
            pl.BlockSpec((H, 1), lambda i: (0, 0)),
            pl.BlockSpec((1, 1), lambda i: (0, 0)),
        ],
        out_specs=[
            pl.BlockSpec((G, H), lambda i: (0, 0)),
            pl.BlockSpec((G, 1), lambda i: (0, 0)),
            pl.BlockSpec((G, 1), lambda i: (0, 0)),
        ],
        out_shape=[
            jax.ShapeDtypeStruct((G, H), jnp.float32),
            jax.ShapeDtypeStruct((G, 1), jnp.float32),
            jax.ShapeDtypeStruct((G, 1), jnp.float32),
        ],
    )(hmid, s1, s2, g, b, batch2, w1, b1, w2, b2, w3, b3)
    return outs[2]


def _pack_tables(u, v):
    def pack(t):
        qf = jnp.clip(jnp.round(t[:, :H] * 1024.0),
                      -32768, 32767).astype(jnp.int32)
        qs = jnp.clip(jnp.round(t[:, H:] * 1024.0),
                      -32768, 32767).astype(jnp.int32)
        p = (qf << 16) | (qs & 0xFFFF)
        return p.reshape(N, 32, FPT).transpose(1, 0, 2).reshape(32, 1, TW)
    return pack(u), pack(v)


def kernel(x, edge_index, batch, edge_attr, lin_in_W, lin_in_b, Wf, bf, Ws,
           bs, gamma, beta, W1, b1, W2, b2, W3, b3):
    src = edge_index[0]
    dst = edge_index[1]
    eat = edge_attr.T
    wd = jnp.concatenate([Wf[:, :H, :], Ws[:, :H, :]], axis=2)
    wsrc = jnp.concatenate([Wf[:, H:2 * H, :], Ws[:, H:2 * H, :]], axis=2)
    we = jnp.concatenate([Wf[:, 2 * H:, :], Ws[:, 2 * H:, :]], axis=2)
    bd = jnp.concatenate([bf, bs], axis=1)
    wef = we[:, :, :H].reshape(L, ED, 32, FPT).transpose(0, 2, 1, 3)
    wes = we[:, :, H:].reshape(L, ED, 32, FPT).transpose(0, 2, 1, 3)
    wet = jnp.concatenate([wef, wes], axis=2)
    wet = jnp.pad(wet, ((0, 0), (0, 0), (0, 0), (0, 16 - FPT)))

    cntp = _cnt_kernel(dst).reshape(16, NBLK, BLK).transpose(1, 0, 2)

    h, u, v = _k0(x, lin_in_W, lin_in_b.reshape(1, H),
                  wd[0], bd[0].reshape(1, 2 * H), wsrc[0])

    s1 = s2 = None
    for l in range(L):
        if l > 0:
            h, u, v = _kapply(h, s1, s2, gamma[l - 1].reshape(1, H),
                              beta[l - 1].reshape(1, H), wd[l],
                              bd[l].reshape(1, 2 * H), wsrc[l])
        ut, vt = _pack_tables(u, v)
        agg_t = _edge_kernel(dst, src, eat, ut, vt, wet[l])
        agg = agg_t.reshape(32, N, FPT).transpose(1, 0, 2).reshape(N, H)
        if l == 0:
            h, s1, s2, cnt = _kstats0(h, agg, cntp)
        else:
            h, s1, s2 = _kstats(h, agg, cnt)

    return _kpool(h, s1, s2, gamma[L - 1].reshape(1, H),
                  beta[L - 1].reshape(1, H), batch.reshape(N, 1),
                  W1, b1.reshape(1, H), W2, b2.reshape(1, H), W3,
                  b3.reshape(1, 1))


# speedup vs baseline: 1.2230x; 1.2230x over previous
import functools

import jax
import jax.numpy as jnp
from jax import lax
from jax.experimental import pallas as pl
from jax.experimental.pallas import tpu as pltpu
from jax.experimental.pallas import tpu_sc as plsc

N = 10000
E = 320000
H = 128
ED = 4
L = 5
G = 64

EPT = E // 16
C = 80
NCHUNK = EPT // C
NPAD = 10240
HH = H // 2
RPT = NPAD // 16
RCH = 128
NRCH = RPT // RCH

BLK = 1000
NBLK = N // BLK

_LN2 = 0.6931471805599453
_LOGC = (-1.7207799231e-02, 1.8497244068e-01, -8.5552603849e-01,
         2.2311249058e+00, -3.6488033354e+00, 4.2045130615e+00,
         -2.0990697238e+00)


def _softplus16(x):
    t = jnp.exp(x)
    y = 1.0 + t
    i = plsc.bitcast(y, jnp.int32)
    e = lax.shift_right_logical(i, 23).astype(jnp.float32) - 127.0
    m = plsc.bitcast(
        jnp.bitwise_or(jnp.bitwise_and(i, 0x007FFFFF), 0x3F800000),
        jnp.float32)
    p = jnp.full((16,), _LOGC[0], jnp.float32)
    for c in _LOGC[1:]:
        p = p * m + c
    r = e * _LN2 + p
    return jnp.where(x > 15.0, x, r)


def _sigmoid16(x):
    return 1.0 / (1.0 + jnp.exp(-x))


EPW = E // 16
CC = 80
NCC = EPW // CC
CE = 256
NCE = E // CE
FPT = 4
TW = N * FPT

_mesh = plsc.VectorSubcoreMesh(core_axis_name="c", subcore_axis_name="s")


def _make_cnt_kernel():
    scratch = [
        pltpu.VMEM((CC,), jnp.int32),
        pltpu.VMEM((1, 5120), jnp.float32),
    ]

    def body(dst_hbm, part_out, dstv, cnt):
        cid = lax.axis_index("c")
        sid = lax.axis_index("s")
        zero16 = jnp.zeros((16,), jnp.float32)
        ones16 = jnp.ones((16,), jnp.float32)
        zi16 = jnp.zeros((16,), jnp.int32)

        @pl.when(cid == 0)
        def _():
            base0 = sid * EPW
            for lo, sz in ((0, 5120), (5120, 5120)):
                def zf(r, _):
                    cnt[0, pl.ds(r * 16, 16)] = zero16
                    return 0
                lax.fori_loop(0, 5120 // 16, zf, 0)

                def chunk(t, _):
                    pltpu.sync_copy(dst_hbm.at[pl.ds(base0 + t * CC, CC)],
                                    dstv)
                    for j in range(CC // 16):
                        d16 = dstv[pl.ds(j * 16, 16)] - lo
                        msk = (d16 >= 0) & (d16 < sz)
                        d16 = jnp.clip(d16, 0, sz - 1)
                        plsc.addupdate_scatter(cnt, [zi16, d16], ones16,
                                               mask=msk)
                    return 0
                lax.fori_loop(0, NCC, chunk, 0)
                pltpu.sync_copy(cnt.at[pl.ds(0, 1), pl.ds(0, sz)],
                                part_out.at[sid, pl.ds(0, 1), pl.ds(lo, sz)])

    return pl.kernel(body,
                     out_type=jax.ShapeDtypeStruct((16, 1, NPAD),
                                                   jnp.float32),
                     mesh=_mesh, scratch_types=scratch,
                     compiler_params=pltpu.CompilerParams(
                         needs_layout_passes=False))


_cnt_kernel = _make_cnt_kernel()


def _make_edge_kernel():
    scratch = [
        pltpu.VMEM((CE,), jnp.int32),
        pltpu.VMEM((CE,), jnp.int32),
        pltpu.VMEM((4, CE), jnp.float32),
        pltpu.VMEM((TW,), jnp.int32),
        pltpu.VMEM((TW,), jnp.int32),
        pltpu.VMEM((TW,), jnp.float32),
        pltpu.VMEM((8, 16), jnp.float32),
        pltpu.SemaphoreType.DMA,
        pltpu.SemaphoreType.DMA,
        pltpu.SemaphoreType.DMA,
    ]

    def body(dst_hbm, src_hbm, eat_hbm, ut_hbm, vt_hbm, wet_hbm,
             agg_out, dstv, srcv, eat, utab, vtab, agg, wet,
             sem_d, sem_s, sem_e):
        cid = lax.axis_index("c")
        sid = lax.axis_index("s")
        w = cid * 16 + sid

        zero16 = jnp.zeros((16,), jnp.float32)
        zi16 = jnp.zeros((16,), jnp.int32)

        pltpu.sync_copy(ut_hbm.at[w, 0], utab)
        pltpu.sync_copy(vt_hbm.at[w, 0], vtab)
        pltpu.sync_copy(wet_hbm.at[w], wet)

        def zf(r, _):
            agg[pl.ds(r * 16, 16)] = zero16
            return 0
        lax.fori_loop(0, TW // 16, zf, 0)

        wf = [[wet[k, pl.ds(0, 16)][c] for c in range(FPT)] for k in range(4)]
        ws = [[wet[4 + k, pl.ds(0, 16)][c] for c in range(FPT)]
              for k in range(4)]

        def chunk(t, _):
            b = t * CE
            cpd = pltpu.async_copy(dst_hbm.at[pl.ds(b, CE)], dstv, sem_d)
            cps = pltpu.async_copy(src_hbm.at[pl.ds(b, CE)], srcv, sem_s)
            cpe = pltpu.async_copy(eat_hbm.at[pl.ds(0, 4), pl.ds(b, CE)],
                                   eat, sem_e)
            cpd.wait()
            cps.wait()
            cpe.wait()
            for j in range(CE // 16):
                o = j * 16
                d16 = dstv[pl.ds(o, 16)]
                s16 = srcv[pl.ds(o, 16)]
                ea = [eat[k, pl.ds(o, 16)] for k in range(4)]
                d4 = d16 * FPT
                s4 = s16 * FPT
                for c in range(FPT):
                    ug = plsc.load_gather(utab, [d4 + c])
                    vg = plsc.load_gather(vtab, [s4 + c])
                    ufi = lax.shift_right_arithmetic(ug, 16)
                    usi = lax.shift_right_arithmetic(lax.shift_left(ug, 16),
                                                     16)
                    vfi = lax.shift_right_arithmetic(vg, 16)
                    vsi = lax.shift_right_arithmetic(lax.shift_left(vg, 16),
                                                     16)
                    lf = (ufi + vfi).astype(jnp.float32) * 9.765625e-4
                    ls = (usi + vsi).astype(jnp.float32) * 9.765625e-4
                    for k in range(4):
                        lf = lf + ea[k] * wf[k][c]
                        ls = ls + ea[k] * ws[k][c]
                    msg = _sigmoid16(lf) * _softplus16(ls)
                    plsc.addupdate_scatter(agg, [d4 + c], msg)
            return 0
        lax.fori_loop(0, NCE, chunk, 0)

        pltpu.sync_copy(agg, agg_out.at[w, 0])

    return pl.kernel(body,
                     out_type=jax.ShapeDtypeStruct((32, 1, TW), jnp.float32),
                     mesh=_mesh, scratch_types=scratch,
                     compiler_params=pltpu.CompilerParams(
                         needs_layout_passes=False))


_edge_kernel = _make_edge_kernel()


def _k0_body(x_ref, win_ref, bin_ref, wd_ref, bd_ref, wsrc_ref,
             h_ref, u_ref, v_ref):
    h = jnp.dot(x_ref[...], win_ref[...],
                preferred_element_type=jnp.float32) + bin_ref[...]
    h_ref[...] = h
    u_ref[...] = jnp.dot(h, wd_ref[...],
                         preferred_element_type=jnp.float32) + bd_ref[...]
    v_ref[...] = jnp.dot(h, wsrc_ref[...], preferred_element_type=jnp.float32)


def _k0(x, win, binb, wd, bd, wsrc):
    return pl.pallas_call(
        _k0_body,
        grid=(NBLK,),
        in_specs=[
            pl.BlockSpec((BLK, H), lambda i: (i, 0)),
            pl.BlockSpec((H, H), lambda i: (0, 0)),
            pl.BlockSpec((1, H), lambda i: (0, 0)),
            pl.BlockSpec((H, 2 * H), lambda i: (0, 0)),
            pl.BlockSpec((1, 2 * H), lambda i: (0, 0)),
            pl.BlockSpec((H, 2 * H), lambda i: (0, 0)),
        ],
        out_specs=[
            pl.BlockSpec((BLK, H), lambda i: (i, 0)),
            pl.BlockSpec((BLK, 2 * H), lambda i: (i, 0)),
            pl.BlockSpec((BLK, 2 * H), lambda i: (i, 0)),
        ],
        out_shape=[
            jax.ShapeDtypeStruct((N, H), jnp.float32),
            jax.ShapeDtypeStruct((N, 2 * H), jnp.float32),
            jax.ShapeDtypeStruct((N, 2 * H), jnp.float32),
        ],
    )(x, win, binb, wd, bd, wsrc)


def _kstats0_body(h_ref, agg_ref, cntp_ref, hmid_ref, s1_ref, s2_ref,
                  cntc_ref):
    i = pl.program_id(0)
    cs = cntp_ref[0]
    c = jnp.maximum(jnp.sum(cs, axis=0), 1.0).reshape(BLK, 1)
    cntc_ref[...] = c
    hm = h_ref[...] + agg_ref[...] / c
    hmid_ref[...] = hm

    @pl.when(i == 0)
    def _():
        s1_ref[...] = jnp.sum(hm, axis=0, keepdims=True)
        s2_ref[...] = jnp.sum(hm * hm, axis=0, keepdims=True)

    @pl.when(i != 0)
    def _():
        s1_ref[...] += jnp.sum(hm, axis=0, keepdims=True)
        s2_ref[...] += jnp.sum(hm * hm, axis=0, keepdims=True)


def _kstats0(h, agg, cntp):
    return pl.pallas_call(
        _kstats0_body,
        grid=(NBLK,),
        in_specs=[
            pl.BlockSpec((BLK, H), lambda i: (i, 0)),
            pl.BlockSpec((BLK, H), lambda i: (i, 0)),
            pl.BlockSpec((1, 16, BLK), lambda i: (i, 0, 0)),
        ],
        out_specs=[
            pl.BlockSpec((BLK, H), lambda i: (i, 0)),
            pl.BlockSpec((1, H), lambda i: (0, 0)),
            pl.BlockSpec((1, H), lambda i: (0, 0)),
            pl.BlockSpec((BLK, 1), lambda i: (i, 0)),
        ],
        out_shape=[
            jax.ShapeDtypeStruct((N, H), jnp.float32),
            jax.ShapeDtypeStruct((1, H), jnp.float32),
            jax.ShapeDtypeStruct((1, H), jnp.float32),
            jax.ShapeDtypeStruct((N, 1), jnp.float32),
        ],
    )(h, agg, cntp)


def _kstats_body(h_ref, agg_ref, cnt_ref, hmid_ref, s1_ref, s2_ref):
    hm = h_ref[...] + agg_ref[...] / cnt_ref[...]
    hmid_ref[...] = hm
    i = pl.program_id(0)

    @pl.when(i == 0)
    def _():
        s1_ref[...] = jnp.sum(hm, axis=0, keepdims=True)
        s2_ref[...] = jnp.sum(hm * hm, axis=0, keepdims=True)

    @pl.when(i != 0)
    def _():
        s1_ref[...] += jnp.sum(hm, axis=0, keepdims=True)
        s2_ref[...] += jnp.sum(hm * hm, axis=0, keepdims=True)


def _kstats(h, agg, cnt):
    return pl.pallas_call(
        _kstats_body,
        grid=(NBLK,),
        in_specs=[
            pl.BlockSpec((BLK, H), lambda i: (i, 0)),
            pl.BlockSpec((BLK, H), lambda i: (i, 0)),
            pl.BlockSpec((BLK, 1), lambda i: (i, 0)),
        ],
        out_specs=[
            pl.BlockSpec((BLK, H), lambda i: (i, 0)),
            pl.BlockSpec((1, H), lambda i: (0, 0)),
            pl.BlockSpec((1, H), lambda i: (0, 0)),
        ],
        out_shape=[
            jax.ShapeDtypeStruct((N, H), jnp.float32),
            jax.ShapeDtypeStruct((1, H), jnp.float32),
            jax.ShapeDtypeStruct((1, H), jnp.float32),
        ],
    )(h, agg, cnt)


def _bn_relu(hmid, s1, s2, g, b):
    mu = s1 / float(N)
    var = s2 / float(N) - mu * mu
    return jnp.maximum(g * (hmid - mu) * lax.rsqrt(var + 1e-5) + b, 0.0)


def _kapply_body(hmid_ref, s1_ref, s2_ref, g_ref, b_ref, wd_ref, bd_ref,
                 wsrc_ref, hn_ref, u_ref, v_ref):
    hn = _bn_relu(hmid_ref[...], s1_ref[...], s2_ref[...], g_ref[...],
                  b_ref[...])
    hn_ref[...] = hn
    u_ref[...] = jnp.dot(hn, wd_ref[...],
                         preferred_element_type=jnp.float32) + bd_ref[...]
    v_ref[...] = jnp.dot(hn, wsrc_ref[...], preferred_element_type=jnp.float32)


def _kapply(hmid, s1, s2, g, b, wd, bd, wsrc):
    return pl.pallas_call(
        _kapply_body,
        grid=(NBLK,),
        in_specs=[
            pl.BlockSpec((BLK, H), lambda i: (i, 0)),
            pl.BlockSpec((1, H), lambda i: (0, 0)),
            pl.BlockSpec((1, H), lambda i: (0, 0)),
            pl.BlockSpec((1, H), lambda i: (0, 0)),
            pl.BlockSpec((1, H), lambda i: (0, 0)),
            pl.BlockSpec((H, 2 * H), lambda i: (0, 0)),
            pl.BlockSpec((1, 2 * H), lambda i: (0, 0)),
            pl.BlockSpec((H, 2 * H), lambda i: (0, 0)),
        ],
        out_specs=[
            pl.BlockSpec((BLK, H), lambda i: (i, 0)),
            pl.BlockSpec((BLK, 2 * H), lambda i: (i, 0)),
            pl.BlockSpec((BLK, 2 * H), lambda i: (i, 0)),
        ],
        out_shape=[
            jax.ShapeDtypeStruct((N, H), jnp.float32),
            jax.ShapeDtypeStruct((N, 2 * H), jnp.float32),
            jax.ShapeDtypeStruct((N, 2 * H), jnp.float32),
        ],
    )(hmid, s1, s2, g, b, wd, bd, wsrc)


def _kpool_body(hmid_ref, s1_ref, s2_ref, g_ref, b_ref, batch_ref,
                w1_ref, b1_ref, w2_ref, b2_ref, w3_ref, b3_ref,
                pooled_ref, gcnt_ref, out_ref):
    hn = _bn_relu(hmid_ref[...], s1_ref[...], s2_ref[...], g_ref[...],
                  b_ref[...])
    bt = batch_ref[...]
    gids = lax.broadcasted_iota(jnp.int32, (G, BLK), 0)
    m = (gids == bt[:, 0][None, :]).astype(jnp.float32)
    pp = jnp.dot(m, hn, preferred_element_type=jnp.float32)
    gc = jnp.sum(m, axis=1, keepdims=True)
    i = pl.program_id(0)

    @pl.when(i == 0)
    def _():
        pooled_ref[...] = pp
        gcnt_ref[...] = gc

    @pl.when(i != 0)
    def _():
        pooled_ref[...] += pp
        gcnt_ref[...] += gc

    @pl.when(i == NBLK - 1)
    def _():
        p = pooled_ref[...] / jnp.maximum(gcnt_ref[...], 1.0)
        o = jnp.maximum(jnp.dot(p, w1_ref[...],
                                preferred_element_type=jnp.float32)
                        + b1_ref[...], 0.0)
        o = jnp.maximum(jnp.dot(o, w2_ref[...],
                                preferred_element_type=jnp.float32)
                        + b2_ref[...], 0.0)
        out_ref[...] = jnp.dot(o, w3_ref[...],
                               preferred_element_type=jnp.float32) + b3_ref[...]


def _kpool(hmid, s1, s2, g, b, batch2, w1, b1, w2, b2, w3, b3):
    outs = pl.pallas_call(
        _kpool_body,
        grid=(NBLK,),
        in_specs=[
            pl.BlockSpec((BLK, H), lambda i: (i, 0)),
            pl.BlockSpec((1, H), lambda i: (0, 0)),
            pl.BlockSpec((1, H), lambda i: (0, 0)),
            pl.BlockSpec((1, H), lambda i: (0, 0)),
            pl.BlockSpec((1, H), lambda i: (0, 0)),
            pl.BlockSpec((BLK, 1), lambda i: (i, 0)),
            pl.BlockSpec((H, H), lambda i: (0, 0)),
            pl.BlockSpec((1, H), lambda i: (0, 0)),
            pl.BlockSpec((H, H), lambda i: (0, 0)),
            pl.BlockSpec((1, H), lambda i: (0, 0)),
            pl.BlockSpec((H, 1), lambda i: (0, 0)),
            pl.BlockSpec((1, 1), lambda i: (0, 0)),
        ],
        out_specs=[
            pl.BlockSpec((G, H), lambda i: (0, 0)),
            pl.BlockSpec((G, 1), lambda i: (0, 0)),
            pl.BlockSpec((G, 1), lambda i: (0, 0)),
        ],
        out_shape=[
            jax.ShapeDtypeStruct((G, H), jnp.float32),
            jax.ShapeDtypeStruct((G, 1), jnp.float32),
            jax.ShapeDtypeStruct((G, 1), jnp.float32),
        ],
    )(hmid, s1, s2, g, b, batch2, w1, b1, w2, b2, w3, b3)
    return outs[2]


def _pack_tables(u, v):
    def pack(t):
        qf = jnp.clip(jnp.round(t[:, :H] * 1024.0),
                      -32768, 32767).astype(jnp.int32)
        qs = jnp.clip(jnp.round(t[:, H:] * 1024.0),
                      -32768, 32767).astype(jnp.int32)
        p = (qf << 16) | (qs & 0xFFFF)
        return p.reshape(N, 32, FPT).transpose(1, 0, 2).reshape(32, 1, TW)
    return pack(u), pack(v)


def kernel(x, edge_index, batch, edge_attr, lin_in_W, lin_in_b, Wf, bf, Ws,
           bs, gamma, beta, W1, b1, W2, b2, W3, b3):
    src = edge_index[0]
    dst = edge_index[1]
    eat = edge_attr.T
    wd = jnp.concatenate([Wf[:, :H, :], Ws[:, :H, :]], axis=2)
    wsrc = jnp.concatenate([Wf[:, H:2 * H, :], Ws[:, H:2 * H, :]], axis=2)
    we = jnp.concatenate([Wf[:, 2 * H:, :], Ws[:, 2 * H:, :]], axis=2)
    bd = jnp.concatenate([bf, bs], axis=1)
    wef = we[:, :, :H].reshape(L, ED, 32, FPT).transpose(0, 2, 1, 3)
    wes = we[:, :, H:].reshape(L, ED, 32, FPT).transpose(0, 2, 1, 3)
    wet = jnp.concatenate([wef, wes], axis=2)
    wet = jnp.pad(wet, ((0, 0), (0, 0), (0, 0), (0, 16 - FPT)))

    cntp = _cnt_kernel(dst).reshape(16, NPAD)[:, :N].reshape(16, NBLK, BLK).transpose(1, 0, 2)

    h, u, v = _k0(x, lin_in_W, lin_in_b.reshape(1, H),
                  wd[0], bd[0].reshape(1, 2 * H), wsrc[0])

    s1 = s2 = None
    for l in range(L):
        if l > 0:
            h, u, v = _kapply(h, s1, s2, gamma[l - 1].reshape(1, H),
                              beta[l - 1].reshape(1, H), wd[l],
                              bd[l].reshape(1, 2 * H), wsrc[l])
        ut, vt = _pack_tables(u, v)
        agg_t = _edge_kernel(dst, src, eat, ut, vt, wet[l])
        agg = agg_t.reshape(32, N, FPT).transpose(1, 0, 2).reshape(N, H)
        if l == 0:
            h, s1, s2, cnt = _kstats0(h, agg, cntp)
        else:
            h, s1, s2 = _kstats(h, agg, cnt)

    return _kpool(h, s1, s2, gamma[L - 1].reshape(1, H),
                  beta[L - 1].reshape(1, H), batch.reshape(N, 1),
                  W1, b1.reshape(1, H), W2, b2.reshape(1, H), W3,
                  b3.reshape(1, 1))


# speedup vs baseline: 2.1266x; 1.7388x over previous
import functools

import jax
import jax.numpy as jnp
from jax import lax
from jax.experimental import pallas as pl
from jax.experimental.pallas import tpu as pltpu
from jax.experimental.pallas import tpu_sc as plsc

N = 10000
E = 320000
H = 128
ED = 4
L = 5
G = 64

EPT = E // 16
C = 80
NCHUNK = EPT // C
NPAD = 10240
HH = H // 2
RPT = NPAD // 16
RCH = 128
NRCH = RPT // RCH

BLK = 1000
NBLK = N // BLK

_LN2 = 0.6931471805599453
_LOGC = (-1.7207799231e-02, 1.8497244068e-01, -8.5552603849e-01,
         2.2311249058e+00, -3.6488033354e+00, 4.2045130615e+00,
         -2.0990697238e+00)


def _softplus16(x):
    t = jnp.exp(x)
    y = 1.0 + t
    i = plsc.bitcast(y, jnp.int32)
    e = lax.shift_right_logical(i, 23).astype(jnp.float32) - 127.0
    m = plsc.bitcast(
        jnp.bitwise_or(jnp.bitwise_and(i, 0x007FFFFF), 0x3F800000),
        jnp.float32)
    p = jnp.full((16,), _LOGC[0], jnp.float32)
    for c in _LOGC[1:]:
        p = p * m + c
    r = e * _LN2 + p
    return jnp.where(x > 15.0, x, r)


def _sigmoid16(x):
    return 1.0 / (1.0 + jnp.exp(-x))


EPW = E // 16
CC = 80
NCC = EPW // CC
CE = 256
NCE = E // CE
FPT = 4
TW = N * FPT

_mesh = plsc.VectorSubcoreMesh(core_axis_name="c", subcore_axis_name="s")


def _make_cnt_kernel():
    scratch = [
        pltpu.VMEM((CC,), jnp.int32),
        pltpu.VMEM((1, 5120), jnp.float32),
    ]

    def body(dst_hbm, part_out, dstv, cnt):
        cid = lax.axis_index("c")
        sid = lax.axis_index("s")
        zero16 = jnp.zeros((16,), jnp.float32)
        ones16 = jnp.ones((16,), jnp.float32)
        zi16 = jnp.zeros((16,), jnp.int32)

        @pl.when(cid == 0)
        def _():
            base0 = sid * EPW
            for lo, sz in ((0, 5120), (5120, 5120)):
                def zf(r, _):
                    cnt[0, pl.ds(r * 16, 16)] = zero16
                    return 0
                lax.fori_loop(0, 5120 // 16, zf, 0)

                def chunk(t, _):
                    pltpu.sync_copy(dst_hbm.at[pl.ds(base0 + t * CC, CC)],
                                    dstv)
                    for j in range(CC // 16):
                        d16 = dstv[pl.ds(j * 16, 16)] - lo
                        msk = (d16 >= 0) & (d16 < sz)
                        d16 = jnp.clip(d16, 0, sz - 1)
                        plsc.addupdate_scatter(cnt, [zi16, d16], ones16,
                                               mask=msk)
                    return 0
                lax.fori_loop(0, NCC, chunk, 0)
                pltpu.sync_copy(cnt.at[pl.ds(0, 1), pl.ds(0, sz)],
                                part_out.at[sid, pl.ds(0, 1), pl.ds(lo, sz)])

    return pl.kernel(body,
                     out_type=jax.ShapeDtypeStruct((16, 1, NPAD),
                                                   jnp.float32),
                     mesh=_mesh, scratch_types=scratch,
                     compiler_params=pltpu.CompilerParams(
                         needs_layout_passes=False))


_cnt_kernel = _make_cnt_kernel()


def _make_edge_kernel():
    scratch = [
        pltpu.VMEM((CE,), jnp.int32),
        pltpu.VMEM((CE,), jnp.int32),
        pltpu.VMEM((4, CE), jnp.float32),
        pltpu.VMEM((TW,), jnp.int32),
        pltpu.VMEM((TW,), jnp.int32),
        pltpu.VMEM((TW,), jnp.float32),
        pltpu.VMEM((8, 16), jnp.float32),
        pltpu.SemaphoreType.DMA,
        pltpu.SemaphoreType.DMA,
        pltpu.SemaphoreType.DMA,
    ]

    def body(dst_hbm, src_hbm, eat_hbm, ut_hbm, vt_hbm, wet_hbm,
             agg_out, dstv, srcv, eat, utab, vtab, agg, wet,
             sem_d, sem_s, sem_e):
        cid = lax.axis_index("c")
        sid = lax.axis_index("s")
        w = cid * 16 + sid

        zero16 = jnp.zeros((16,), jnp.float32)
        zi16 = jnp.zeros((16,), jnp.int32)

        pltpu.sync_copy(ut_hbm.at[w, 0], utab)
        pltpu.sync_copy(vt_hbm.at[w, 0], vtab)
        pltpu.sync_copy(wet_hbm.at[w], wet)

        def zf(r, _):
            agg[pl.ds(r * 16, 16)] = zero16
            return 0
        lax.fori_loop(0, TW // 16, zf, 0)

        wf = [[wet[k, pl.ds(0, 16)][c] for c in range(FPT)] for k in range(4)]
        ws = [[wet[4 + k, pl.ds(0, 16)][c] for c in range(FPT)]
              for k in range(4)]

        def chunk(t, _):
            b = t * CE
            cpd = pltpu.async_copy(dst_hbm.at[pl.ds(b, CE)], dstv, sem_d)
            cps = pltpu.async_copy(src_hbm.at[pl.ds(b, CE)], srcv, sem_s)
            cpe = pltpu.async_copy(eat_hbm.at[pl.ds(0, 4), pl.ds(b, CE)],
                                   eat, sem_e)
            cpd.wait()
            cps.wait()
            cpe.wait()

            @plsc.parallel_loop(0, CE // 16, 1, unroll=4)
            def _(j):
                o = j * 16
                d16 = dstv[pl.ds(o, 16)]
                s16 = srcv[pl.ds(o, 16)]
                ea = [eat[k, pl.ds(o, 16)] for k in range(4)]
                d4 = d16 * FPT
                s4 = s16 * FPT
                for c in range(FPT):
                    ug = plsc.load_gather(utab, [d4 + c])
                    vg = plsc.load_gather(vtab, [s4 + c])
                    ufi = lax.shift_right_arithmetic(ug, 16)
                    usi = lax.shift_right_arithmetic(lax.shift_left(ug, 16),
                                                     16)
                    vfi = lax.shift_right_arithmetic(vg, 16)
                    vsi = lax.shift_right_arithmetic(lax.shift_left(vg, 16),
                                                     16)
                    lf = (ufi + vfi).astype(jnp.float32) * 9.765625e-4
                    ls = (usi + vsi).astype(jnp.float32) * 9.765625e-4
                    for k in range(4):
                        lf = lf + ea[k] * wf[k][c]
                        ls = ls + ea[k] * ws[k][c]
                    msg = _sigmoid16(lf) * _softplus16(ls)
                    plsc.addupdate_scatter(agg, [d4 + c], msg)
            return 0
        lax.fori_loop(0, NCE, chunk, 0)

        pltpu.sync_copy(agg, agg_out.at[w, 0])

    return pl.kernel(body,
                     out_type=jax.ShapeDtypeStruct((32, 1, TW), jnp.float32),
                     mesh=_mesh, scratch_types=scratch,
                     compiler_params=pltpu.CompilerParams(
                         needs_layout_passes=False))


_edge_kernel = _make_edge_kernel()


def _k0_body(x_ref, win_ref, bin_ref, wd_ref, bd_ref, wsrc_ref,
             h_ref, u_ref, v_ref):
    h = jnp.dot(x_ref[...], win_ref[...],
                preferred_element_type=jnp.float32) + bin_ref[...]
    h_ref[...] = h
    u_ref[...] = jnp.dot(h, wd_ref[...],
                         preferred_element_type=jnp.float32) + bd_ref[...]
    v_ref[...] = jnp.dot(h, wsrc_ref[...], preferred_element_type=jnp.float32)


def _k0(x, win, binb, wd, bd, wsrc):
    return pl.pallas_call(
        _k0_body,
        grid=(NBLK,),
        in_specs=[
            pl.BlockSpec((BLK, H), lambda i: (i, 0)),
            pl.BlockSpec((H, H), lambda i: (0, 0)),
            pl.BlockSpec((1, H), lambda i: (0, 0)),
            pl.BlockSpec((H, 2 * H), lambda i: (0, 0)),
            pl.BlockSpec((1, 2 * H), lambda i: (0, 0)),
            pl.BlockSpec((H, 2 * H), lambda i: (0, 0)),
        ],
        out_specs=[
            pl.BlockSpec((BLK, H), lambda i: (i, 0)),
            pl.BlockSpec((BLK, 2 * H), lambda i: (i, 0)),
            pl.BlockSpec((BLK, 2 * H), lambda i: (i, 0)),
        ],
        out_shape=[
            jax.ShapeDtypeStruct((N, H), jnp.float32),
            jax.ShapeDtypeStruct((N, 2 * H), jnp.float32),
            jax.ShapeDtypeStruct((N, 2 * H), jnp.float32),
        ],
    )(x, win, binb, wd, bd, wsrc)


def _kstats0_body(h_ref, agg_ref, cntp_ref, hmid_ref, s1_ref, s2_ref,
                  cntc_ref):
    i = pl.program_id(0)
    cs = cntp_ref[0]
    c = jnp.maximum(jnp.sum(cs, axis=0), 1.0).reshape(BLK, 1)
    cntc_ref[...] = c
    hm = h_ref[...] + agg_ref[...] / c
    hmid_ref[...] = hm

    @pl.when(i == 0)
    def _():
        s1_ref[...] = jnp.sum(hm, axis=0, keepdims=True)
        s2_ref[...] = jnp.sum(hm * hm, axis=0, keepdims=True)

    @pl.when(i != 0)
    def _():
        s1_ref[...] += jnp.sum(hm, axis=0, keepdims=True)
        s2_ref[...] += jnp.sum(hm * hm, axis=0, keepdims=True)


def _kstats0(h, agg, cntp):
    return pl.pallas_call(
        _kstats0_body,
        grid=(NBLK,),
        in_specs=[
            pl.BlockSpec((BLK, H), lambda i: (i, 0)),
            pl.BlockSpec((BLK, H), lambda i: (i, 0)),
            pl.BlockSpec((1, 16, BLK), lambda i: (i, 0, 0)),
        ],
        out_specs=[
            pl.BlockSpec((BLK, H), lambda i: (i, 0)),
            pl.BlockSpec((1, H), lambda i: (0, 0)),
            pl.BlockSpec((1, H), lambda i: (0, 0)),
            pl.BlockSpec((BLK, 1), lambda i: (i, 0)),
        ],
        out_shape=[
            jax.ShapeDtypeStruct((N, H), jnp.float32),
            jax.ShapeDtypeStruct((1, H), jnp.float32),
            jax.ShapeDtypeStruct((1, H), jnp.float32),
            jax.ShapeDtypeStruct((N, 1), jnp.float32),
        ],
    )(h, agg, cntp)


def _kstats_body(h_ref, agg_ref, cnt_ref, hmid_ref, s1_ref, s2_ref):
    hm = h_ref[...] + agg_ref[...] / cnt_ref[...]
    hmid_ref[...] = hm
    i = pl.program_id(0)

    @pl.when(i == 0)
    def _():
        s1_ref[...] = jnp.sum(hm, axis=0, keepdims=True)
        s2_ref[...] = jnp.sum(hm * hm, axis=0, keepdims=True)

    @pl.when(i != 0)
    def _():
        s1_ref[...] += jnp.sum(hm, axis=0, keepdims=True)
        s2_ref[...] += jnp.sum(hm * hm, axis=0, keepdims=True)


def _kstats(h, agg, cnt):
    return pl.pallas_call(
        _kstats_body,
        grid=(NBLK,),
        in_specs=[
            pl.BlockSpec((BLK, H), lambda i: (i, 0)),
            pl.BlockSpec((BLK, H), lambda i: (i, 0)),
            pl.BlockSpec((BLK, 1), lambda i: (i, 0)),
        ],
        out_specs=[
            pl.BlockSpec((BLK, H), lambda i: (i, 0)),
            pl.BlockSpec((1, H), lambda i: (0, 0)),
            pl.BlockSpec((1, H), lambda i: (0, 0)),
        ],
        out_shape=[
            jax.ShapeDtypeStruct((N, H), jnp.float32),
            jax.ShapeDtypeStruct((1, H), jnp.float32),
            jax.ShapeDtypeStruct((1, H), jnp.float32),
        ],
    )(h, agg, cnt)


def _bn_relu(hmid, s1, s2, g, b):
    mu = s1 / float(N)
    var = s2 / float(N) - mu * mu
    return jnp.maximum(g * (hmid - mu) * lax.rsqrt(var + 1e-5) + b, 0.0)


def _kapply_body(hmid_ref, s1_ref, s2_ref, g_ref, b_ref, wd_ref, bd_ref,
                 wsrc_ref, hn_ref, u_ref, v_ref):
    hn = _bn_relu(hmid_ref[...], s1_ref[...], s2_ref[...], g_ref[...],
                  b_ref[...])
    hn_ref[...] = hn
    u_ref[...] = jnp.dot(hn, wd_ref[...],
                         preferred_element_type=jnp.float32) + bd_ref[...]
    v_ref[...] = jnp.dot(hn, wsrc_ref[...], preferred_element_type=jnp.float32)


def _kapply(hmid, s1, s2, g, b, wd, bd, wsrc):
    return pl.pallas_call(
        _kapply_body,
        grid=(NBLK,),
        in_specs=[
            pl.BlockSpec((BLK, H), lambda i: (i, 0)),
            pl.BlockSpec((1, H), lambda i: (0, 0)),
            pl.BlockSpec((1, H), lambda i: (0, 0)),
            pl.BlockSpec((1, H), lambda i: (0, 0)),
            pl.BlockSpec((1, H), lambda i: (0, 0)),
            pl.BlockSpec((H, 2 * H), lambda i: (0, 0)),
            pl.BlockSpec((1, 2 * H), lambda i: (0, 0)),
            pl.BlockSpec((H, 2 * H), lambda i: (0, 0)),
        ],
        out_specs=[
            pl.BlockSpec((BLK, H), lambda i: (i, 0)),
            pl.BlockSpec((BLK, 2 * H), lambda i: (i, 0)),
            pl.BlockSpec((BLK, 2 * H), lambda i: (i, 0)),
        ],
        out_shape=[
            jax.ShapeDtypeStruct((N, H), jnp.float32),
            jax.ShapeDtypeStruct((N, 2 * H), jnp.float32),
            jax.ShapeDtypeStruct((N, 2 * H), jnp.float32),
        ],
    )(hmid, s1, s2, g, b, wd, bd, wsrc)


def _kpool_body(hmid_ref, s1_ref, s2_ref, g_ref, b_ref, batch_ref,
                w1_ref, b1_ref, w2_ref, b2_ref, w3_ref, b3_ref,
                pooled_ref, gcnt_ref, out_ref):
    hn = _bn_relu(hmid_ref[...], s1_ref[...], s2_ref[...], g_ref[...],
                  b_ref[...])
    bt = batch_ref[...]
    gids = lax.broadcasted_iota(jnp.int32, (G, BLK), 0)
    m = (gids == bt[:, 0][None, :]).astype(jnp.float32)
    pp = jnp.dot(m, hn, preferred_element_type=jnp.float32)
    gc = jnp.sum(m, axis=1, keepdims=True)
    i = pl.program_id(0)

    @pl.when(i == 0)
    def _():
        pooled_ref[...] = pp
        gcnt_ref[...] = gc

    @pl.when(i != 0)
    def _():
        pooled_ref[...] += pp
        gcnt_ref[...] += gc

    @pl.when(i == NBLK - 1)
    def _():
        p = pooled_ref[...] / jnp.maximum(gcnt_ref[...], 1.0)
        o = jnp.maximum(jnp.dot(p, w1_ref[...],
                                preferred_element_type=jnp.float32)
                        + b1_ref[...], 0.0)
        o = jnp.maximum(jnp.dot(o, w2_ref[...],
                                preferred_element_type=jnp.float32)
                        + b2_ref[...], 0.0)
        out_ref[...] = jnp.dot(o, w3_ref[...],
                               preferred_element_type=jnp.float32) + b3_ref[...]


def _kpool(hmid, s1, s2, g, b, batch2, w1, b1, w2, b2, w3, b3):
    outs = pl.pallas_call(
        _kpool_body,
        grid=(NBLK,),
        in_specs=[
            pl.BlockSpec((BLK, H), lambda i: (i, 0)),
            pl.BlockSpec((1, H), lambda i: (0, 0)),
            pl.BlockSpec((1, H), lambda i: (0, 0)),
            pl.BlockSpec((1, H), lambda i: (0, 0)),
            pl.BlockSpec((1, H), lambda i: (0, 0)),
            pl.BlockSpec((BLK, 1), lambda i: (i, 0)),
            pl.BlockSpec((H, H), lambda i: (0, 0)),
            pl.BlockSpec((1, H), lambda i: (0, 0)),
            pl.BlockSpec((H, H), lambda i: (0, 0)),
            pl.BlockSpec((1, H), lambda i: (0, 0)),
            pl.BlockSpec((H, 1), lambda i: (0, 0)),
            pl.BlockSpec((1, 1), lambda i: (0, 0)),
        ],
        out_specs=[
            pl.BlockSpec((G, H), lambda i: (0, 0)),
            pl.BlockSpec((G, 1), lambda i: (0, 0)),
            pl.BlockSpec((G, 1), lambda i: (0, 0)),
        ],
        out_shape=[
            jax.ShapeDtypeStruct((G, H), jnp.float32),
            jax.ShapeDtypeStruct((G, 1), jnp.float32),
            jax.ShapeDtypeStruct((G, 1), jnp.float32),
        ],
    )(hmid, s1, s2, g, b, batch2, w1, b1, w2, b2, w3, b3)
    return outs[2]


def _pack_tables(u, v):
    def pack(t):
        qf = jnp.clip(jnp.round(t[:, :H] * 1024.0),
                      -32768, 32767).astype(jnp.int32)
        qs = jnp.clip(jnp.round(t[:, H:] * 1024.0),
                      -32768, 32767).astype(jnp.int32)
        p = (qf << 16) | (qs & 0xFFFF)
        return p.reshape(N, 32, FPT).transpose(1, 0, 2).reshape(32, 1, TW)
    return pack(u), pack(v)


def kernel(x, edge_index, batch, edge_attr, lin_in_W, lin_in_b, Wf, bf, Ws,
           bs, gamma, beta, W1, b1, W2, b2, W3, b3):
    src = edge_index[0]
    dst = edge_index[1]
    eat = edge_attr.T
    wd = jnp.concatenate([Wf[:, :H, :], Ws[:, :H, :]], axis=2)
    wsrc = jnp.concatenate([Wf[:, H:2 * H, :], Ws[:, H:2 * H, :]], axis=2)
    we = jnp.concatenate([Wf[:, 2 * H:, :], Ws[:, 2 * H:, :]], axis=2)
    bd = jnp.concatenate([bf, bs], axis=1)
    wef = we[:, :, :H].reshape(L, ED, 32, FPT).transpose(0, 2, 1, 3)
    wes = we[:, :, H:].reshape(L, ED, 32, FPT).transpose(0, 2, 1, 3)
    wet = jnp.concatenate([wef, wes], axis=2)
    wet = jnp.pad(wet, ((0, 0), (0, 0), (0, 0), (0, 16 - FPT)))

    cntp = _cnt_kernel(dst).reshape(16, NPAD)[:, :N].reshape(16, NBLK, BLK).transpose(1, 0, 2)

    h, u, v = _k0(x, lin_in_W, lin_in_b.reshape(1, H),
                  wd[0], bd[0].reshape(1, 2 * H), wsrc[0])

    s1 = s2 = None
    for l in range(L):
        if l > 0:
            h, u, v = _kapply(h, s1, s2, gamma[l - 1].reshape(1, H),
                              beta[l - 1].reshape(1, H), wd[l],
                              bd[l].reshape(1, 2 * H), wsrc[l])
        ut, vt = _pack_tables(u, v)
        agg_t = _edge_kernel(dst, src, eat, ut, vt, wet[l])
        agg = agg_t.reshape(32, N, FPT).transpose(1, 0, 2).reshape(N, H)
        if l == 0:
            h, s1, s2, cnt = _kstats0(h, agg, cntp)
        else:
            h, s1, s2 = _kstats(h, agg, cnt)

    return _kpool(h, s1, s2, gamma[L - 1].reshape(1, H),
                  beta[L - 1].reshape(1, H), batch.reshape(N, 1),
                  W1, b1.reshape(1, H), W2, b2.reshape(1, H), W3,
                  b3.reshape(1, 1))
